# baseline (speedup 1.0000x reference)
import jax
import jax.numpy as jnp
from jax import lax
from jax.experimental import pallas as pl
from jax.experimental.pallas import tpu as pltpu
from jax.experimental.pallas import tpu_sc as plsc

N = 10000
E = 320000
H = 128
ED = 16
G = 64
E_ALL = E + N

NC = 2
NS = 16
NW = NC * NS
C = 96
CHUNKS = 108
E_PAD = NW * CHUNKS * C
N_PAD = 10112

_f32 = jnp.float32
_LOG2E = 1.4426950408889634
_LN2 = 0.6931471805599453


def _take16(v, idx):
    return lax.gather(
        v, idx[:, None],
        dimension_numbers=lax.GatherDimensionNumbers(
            offset_dims=(), collapsed_slice_dims=(0,), start_index_map=(0,)),
        slice_sizes=(1,),
        mode=lax.GatherScatterMode.PROMISE_IN_BOUNDS)


def _exp16(x):
    t = x * _LOG2E
    half = jnp.where(t >= 0, jnp.full((16,), 0.5, _f32),
                     jnp.full((16,), -0.5, _f32))
    n = (t + half).astype(jnp.int32)
    g = (t - n.astype(_f32)) * _LN2
    p = jnp.full((16,), 1.0 / 720.0, _f32)
    p = p * g + 1.0 / 120.0
    p = p * g + 1.0 / 24.0
    p = p * g + 1.0 / 6.0
    p = p * g + 0.5
    p = p * g + 1.0
    p = p * g + 1.0
    nc = jnp.clip(n, -126, 127)
    scale = plsc.bitcast((nc + 127) << 23, _f32)
    return p * scale


def _sc_edge_body(xl_hbm, xr_hbm, e_hbm, src_hbm, dst_hbm, att_hbm,
                  out_hbm, den_hbm,
                  src_b, dst_b, xl_b, xr_b, e_b, s_b, den_b, att_b,
                  acc_out, sem_ix, sem_e, sem_g):
    cid = lax.axis_index("c")
    sid = lax.axis_index("s")
    wid = sid * NC + cid

    def _zrow(i, _):
        for j in range(H // 16):
            xl_b[i, pl.ds(16 * j, 16)] = jnp.zeros((16,), _f32)
        return 0
    lax.fori_loop(0, C, _zrow, 0)

    def _zden(i, _):
        den_b[pl.ds(16 * i, 16)] = jnp.zeros((16,), _f32)
        return 0
    lax.fori_loop(0, N_PAD // 16, _zden, 0)

    rows_per_tile = N_PAD // NS
    row0 = sid * rows_per_tile
    nfull, rem = divmod(rows_per_tile, C)
    for k in range(nfull):
        pltpu.sync_copy(xl_b, acc_out.at[pl.ds(row0 + C * k, C)])
    if rem:
        pltpu.sync_copy(xl_b.at[pl.ds(0, rem)],
                        acc_out.at[pl.ds(row0 + C * nfull, rem)])
    pltpu.sync_copy(att_hbm, att_b)
    plsc.subcore_barrier()

    att_s = [att_b[pl.ds(16 * j, 16)] for j in range(H // 16)]
    iota16 = lax.broadcasted_iota(jnp.int32, (16,), 0)
    lane0 = iota16 == 0

    def _chunk(k, _):
        base = (wid * CHUNKS + k) * C
        cp_e = pltpu.async_copy(e_hbm.at[pl.ds(base, C)], e_b, sem_e)
        cp_s = pltpu.async_copy(src_hbm.at[pl.ds(base, C)], src_b, sem_ix)
        cp_d = pltpu.async_copy(dst_hbm.at[pl.ds(base, C)], dst_b, sem_ix)
        cp_s.wait()
        cp_d.wait()
        cg1 = pltpu.async_copy(xl_hbm.at[src_b], xl_b, sem_g)
        cg2 = pltpu.async_copy(xr_hbm.at[dst_b], xr_b, sem_g)
        cg1.wait()
        cg2.wait()
        cp_e.wait()

        def _edge(i, _):
            xs = []
            acc = jnp.zeros((16,), _f32)
            for j in range(H // 16):
                sl = pl.ds(16 * j, 16)
                xv = xl_b[i, sl]
                xs.append(xv)
                v = xv + xr_b[i, sl] + e_b[i, sl]
                m = jnp.maximum(v, 0.2 * v)
                acc = acc + att_s[j] * m
            total = jnp.sum(acc)
            sv = _exp16(jnp.broadcast_to(total, (16,)))
            sv = jnp.where(base + i < E_ALL, sv, jnp.zeros((16,), _f32))
            plsc.store_scatter(s_b, [jnp.full((16,), i, jnp.int32)], sv,
                               mask=lane0)
            for j in range(H // 16):
                xl_b[i, pl.ds(16 * j, 16)] = xs[j] * sv
            return 0
        lax.fori_loop(0, C, _edge, 0, unroll=2)

        for t in range(C // 16):
            sl = pl.ds(16 * t, 16)
            sk, sc = plsc.sort_key_val(dst_b[sl], s_b[sl])
            for sh in (1, 2, 4, 8):
                idx = jnp.maximum(iota16 - sh, 0)
                m = (iota16 >= sh) & (_take16(sk, idx) == sk)
                sc = sc + jnp.where(m, _take16(sc, idx),
                                    jnp.zeros((16,), _f32))
            nk = _take16(sk, jnp.minimum(iota16 + 1, 15))
            bd = (sk != nk) | (iota16 == 15)
            old = plsc.load_gather(den_b, [sk])
            plsc.store_scatter(den_b, [sk], old + sc, mask=bd)

        pltpu.sync_copy(xl_b, acc_out.at[dst_b], add=True)
        return 0
    lax.fori_loop(0, CHUNKS, _chunk, 0)

    plsc.subcore_barrier()
    pltpu.sync_copy(acc_out.at[pl.ds(row0, rows_per_tile)],
                    out_hbm.at[cid, pl.ds(row0, rows_per_tile)])
    pltpu.sync_copy(den_b, den_hbm.at[cid * NS + sid])


_sc_edge = pl.kernel(
    _sc_edge_body,
    out_type=(
        jax.ShapeDtypeStruct((NC, N_PAD, H), _f32),
        jax.ShapeDtypeStruct((NW, N_PAD), _f32),
    ),
    mesh=plsc.VectorSubcoreMesh(core_axis_name="c", subcore_axis_name="s"),
    compiler_params=pltpu.CompilerParams(needs_layout_passes=False),
    scratch_types=[
        pltpu.VMEM((C,), jnp.int32),
        pltpu.VMEM((C,), jnp.int32),
        pltpu.VMEM((C, H), _f32),
        pltpu.VMEM((C, H), _f32),
        pltpu.VMEM((C, H), _f32),
        pltpu.VMEM((C,), _f32),
        pltpu.VMEM((N_PAD,), _f32),
        pltpu.VMEM((H,), _f32),
        pltpu.VMEM_SHARED((N_PAD, H), _f32),
        pltpu.SemaphoreType.DMA,
        pltpu.SemaphoreType.DMA,
        pltpu.SemaphoreType.DMA,
    ],
)


def _easum_body(ea_ref, out_ref):
    i = pl.program_id(0)

    @pl.when(i == 0)
    def _():
        out_ref[...] = jnp.zeros_like(out_ref)
    out_ref[0, :] += jnp.sum(ea_ref[...], axis=0)


def _easum(ea):
    blk = 4000
    return pl.pallas_call(
        _easum_body,
        grid=(E // blk,),
        in_specs=[pl.BlockSpec((blk, ED), lambda i: (i, 0))],
        out_specs=pl.BlockSpec((1, ED), lambda i: (0, 0)),
        out_shape=jax.ShapeDtypeStruct((1, ED), _f32),
    )(ea)


def _edges_body(ea_ref, we1_ref, we2_ref, e1_ref, e2_ref):
    blk = ea_ref[...]
    e1_ref[...] = jnp.dot(blk, we1_ref[...], preferred_element_type=_f32)
    e2_ref[...] = jnp.dot(blk, we2_ref[...], preferred_element_type=_f32)


def _edges(ea_full, we1, we2):
    blk = 2048
    return pl.pallas_call(
        _edges_body,
        grid=(E_PAD // blk,),
        in_specs=[
            pl.BlockSpec((blk, ED), lambda i: (i, 0)),
            pl.BlockSpec((ED, H), lambda i: (0, 0)),
            pl.BlockSpec((ED, H), lambda i: (0, 0)),
        ],
        out_specs=[
            pl.BlockSpec((blk, H), lambda i: (i, 0)),
            pl.BlockSpec((blk, H), lambda i: (i, 0)),
        ],
        out_shape=[
            jax.ShapeDtypeStruct((E_PAD, H), _f32),
            jax.ShapeDtypeStruct((E_PAD, H), _f32),
        ],
    )(ea_full, we1, we2)


def _nodes_body(x_ref, wl_ref, wr_ref, bl_ref, br_ref, xl_ref, xr_ref):
    xv = x_ref[...]
    xl_ref[...] = jnp.dot(xv, wl_ref[...], preferred_element_type=_f32) \
        + bl_ref[...]
    xr_ref[...] = jnp.dot(xv, wr_ref[...], preferred_element_type=_f32) \
        + br_ref[...]


def _nodes(x, wl, wr, bl, br):
    return pl.pallas_call(
        _nodes_body,
        out_shape=[
            jax.ShapeDtypeStruct((N, H), _f32),
            jax.ShapeDtypeStruct((N, H), _f32),
        ],
    )(x, wl, wr, bl.reshape(1, H), br.reshape(1, H))


def _combine(op_ref, dp_ref, bias_ref):
    o = op_ref[0] + op_ref[1]
    d = jnp.sum(dp_ref[...], axis=0)
    return o[:N] / (d[:N, None] + 1e-16) + bias_ref[...]


def _mid_body(op_ref, dp_ref, bias_ref, g_ref, b_ref, wl_ref, wr_ref,
              bl_ref, br_ref, xl_ref, xr_ref):
    h = _combine(op_ref, dp_ref, bias_ref)
    mu = jnp.mean(h, axis=0, keepdims=True)
    var = jnp.mean((h - mu) ** 2, axis=0, keepdims=True)
    h = (h - mu) / jnp.sqrt(var + 1e-5) * g_ref[...] + b_ref[...]
    h = jnp.maximum(h, 0.0)
    xl_ref[...] = jnp.dot(h, wl_ref[...], preferred_element_type=_f32) \
        + bl_ref[...]
    xr_ref[...] = jnp.dot(h, wr_ref[...], preferred_element_type=_f32) \
        + br_ref[...]


def _mid(op, dp, bias, g, b, wl, wr, bl, br):
    return pl.pallas_call(
        _mid_body,
        out_shape=[
            jax.ShapeDtypeStruct((N, H), _f32),
            jax.ShapeDtypeStruct((N, H), _f32),
        ],
    )(op, dp, bias.reshape(1, H), g.reshape(1, H), b.reshape(1, H),
      wl, wr, bl.reshape(1, H), br.reshape(1, H))


def _post_body(op_ref, dp_ref, bias_ref, batch_ref, fcw_ref, fcb_ref, y_ref):
    h = _combine(op_ref, dp_ref, bias_ref)
    bvec = batch_ref[0, :]
    gi = lax.broadcasted_iota(jnp.int32, (G, N), 0)
    oneh = (gi == bvec[None, :]).astype(_f32)
    ps = jnp.dot(oneh, h, preferred_element_type=_f32,
                 precision=lax.Precision.HIGHEST)
    cnt = jnp.sum(oneh, axis=1)
    pooled = ps / jnp.maximum(cnt, 1.0)[:, None]
    y_ref[...] = jnp.dot(pooled, fcw_ref[...], preferred_element_type=_f32) \
        + fcb_ref[...]


def _post(op, dp, bias, batch2d, fcw, fcb):
    return pl.pallas_call(
        _post_body,
        out_shape=jax.ShapeDtypeStruct((G, 1), _f32),
    )(op, dp, bias.reshape(1, H), batch2d, fcw, fcb.reshape(1, 1))


def kernel(x, edge_attr, params, edge_index, batch):
    p1 = params['conv0']
    p2 = params['conv1']
    pad = E_PAD - E_ALL
    loop = jnp.arange(N, dtype=jnp.int32)
    zpad = jnp.zeros((pad,), jnp.int32)
    src = jnp.concatenate([edge_index[0].astype(jnp.int32), loop, zpad])
    dst = jnp.concatenate([edge_index[1].astype(jnp.int32), loop, zpad])

    ea_sum = _easum(edge_attr)
    ea_mean = ea_sum[0] / E
    ea_full = jnp.concatenate(
        [edge_attr, jnp.broadcast_to(ea_mean, (N, ED)),
         jnp.zeros((pad, ED), _f32)], axis=0)
    e1, e2 = _edges(ea_full, p1['We'], p2['We'])

    xl1, xr1 = _nodes(x, p1['Wl'], p1['Wr'], p1['bl'], p1['br'])
    op1, dp1 = _sc_edge(xl1, xr1, e1, src, dst, p1['att'])
    xl2, xr2 = _mid(op1, dp1, p1['bias'], params['bn0_g'], params['bn0_b'],
                    p2['Wl'], p2['Wr'], p2['bl'], p2['br'])
    op2, dp2 = _sc_edge(xl2, xr2, e2, src, dst, p2['att'])
    batch2d = batch.astype(jnp.int32).reshape(1, N)
    return _post(op2, dp2, p2['bias'], batch2d, params['fc_w'], params['fc_b'])


# speedup vs baseline: 1.1591x; 1.1591x over previous
import jax
import jax.numpy as jnp
from jax import lax
from jax.experimental import pallas as pl
from jax.experimental.pallas import tpu as pltpu
from jax.experimental.pallas import tpu_sc as plsc

N = 10000
E = 320000
H = 128
ED = 16
G = 64
E_ALL = E + N

NC = 2
NS = 16
NW = NC * NS
C = 96
CHUNKS = 108
E_PAD = NW * CHUNKS * C
N_PAD = 10112

_f32 = jnp.float32
_LOG2E = 1.4426950408889634
_LN2 = 0.6931471805599453


def _take16(v, idx):
    return lax.gather(
        v, idx[:, None],
        dimension_numbers=lax.GatherDimensionNumbers(
            offset_dims=(), collapsed_slice_dims=(0,), start_index_map=(0,)),
        slice_sizes=(1,),
        mode=lax.GatherScatterMode.PROMISE_IN_BOUNDS)


def _exp16(x):
    t = x * _LOG2E
    half = jnp.where(t >= 0, jnp.full((16,), 0.5, _f32),
                     jnp.full((16,), -0.5, _f32))
    n = (t + half).astype(jnp.int32)
    g = (t - n.astype(_f32)) * _LN2
    p = jnp.full((16,), 1.0 / 720.0, _f32)
    p = p * g + 1.0 / 120.0
    p = p * g + 1.0 / 24.0
    p = p * g + 1.0 / 6.0
    p = p * g + 0.5
    p = p * g + 1.0
    p = p * g + 1.0
    nc = jnp.clip(n, -126, 127)
    scale = plsc.bitcast((nc + 127) << 23, _f32)
    return p * scale


def _sc_edge_body(xl_hbm, xr_hbm, e_hbm, src_hbm, dst_hbm, att_hbm,
                  out_hbm, den_hbm,
                  src_b, dst_b, xl_b, xr_b, e_b, s_b, den_b, att_b,
                  acc_out, sem_ix, sem_e, sem_g):
    cid = lax.axis_index("c")
    sid = lax.axis_index("s")
    wid = sid * NC + cid

    def _zrow(i, _):
        for j in range(H // 16):
            xl_b[i, pl.ds(16 * j, 16)] = jnp.zeros((16,), _f32)
        return 0
    lax.fori_loop(0, C, _zrow, 0)

    def _zden(i, _):
        den_b[pl.ds(16 * i, 16)] = jnp.zeros((16,), _f32)
        return 0
    lax.fori_loop(0, N_PAD // 16, _zden, 0)

    rows_per_tile = N_PAD // NS
    row0 = sid * rows_per_tile
    nfull, rem = divmod(rows_per_tile, C)
    for k in range(nfull):
        pltpu.sync_copy(xl_b, acc_out.at[pl.ds(row0 + C * k, C)])
    if rem:
        pltpu.sync_copy(xl_b.at[pl.ds(0, rem)],
                        acc_out.at[pl.ds(row0 + C * nfull, rem)])
    pltpu.sync_copy(att_hbm, att_b)
    plsc.subcore_barrier()

    att_s = [att_b[pl.ds(16 * j, 16)] for j in range(H // 16)]
    iota16 = lax.broadcasted_iota(jnp.int32, (16,), 0)
    lane0 = iota16 == 0

    def _chunk(k, _):
        base = (wid * CHUNKS + k) * C
        cp_e = pltpu.async_copy(e_hbm.at[pl.ds(base, C)], e_b, sem_e)
        cp_s = pltpu.async_copy(src_hbm.at[pl.ds(base, C)], src_b, sem_ix)
        cp_d = pltpu.async_copy(dst_hbm.at[pl.ds(base, C)], dst_b, sem_ix)
        cp_s.wait()
        cp_d.wait()
        cg1 = pltpu.async_copy(xl_hbm.at[src_b], xl_b, sem_g)
        cg2 = pltpu.async_copy(xr_hbm.at[dst_b], xr_b, sem_g)
        cg1.wait()
        cg2.wait()
        cp_e.wait()

        def _edge(i, _):
            acc = jnp.zeros((16,), _f32)
            for j in range(H // 16):
                sl = pl.ds(16 * j, 16)
                v = xl_b[i, sl] + xr_b[i, sl] + e_b[i, sl]
                m = jnp.maximum(v, 0.2 * v)
                acc = acc + att_s[j] * m
            total = jnp.sum(acc)
            plsc.store_scatter(s_b, [jnp.full((16,), i, jnp.int32)],
                               jnp.broadcast_to(total, (16,)), mask=lane0)
            return 0
        lax.fori_loop(0, C, _edge, 0, unroll=2)

        for t in range(C // 16):
            sl = pl.ds(16 * t, 16)
            gid = base + 16 * t + iota16
            s16 = jnp.where(gid < E_ALL, _exp16(s_b[sl]),
                            jnp.zeros((16,), _f32))
            s_b[sl] = s16
            sk, sc = plsc.sort_key_val(dst_b[sl], s16)
            for sh in (1, 2, 4, 8):
                idx = jnp.maximum(iota16 - sh, 0)
                m = (iota16 >= sh) & (_take16(sk, idx) == sk)
                sc = sc + jnp.where(m, _take16(sc, idx),
                                    jnp.zeros((16,), _f32))
            nk = _take16(sk, jnp.minimum(iota16 + 1, 15))
            bd = (sk != nk) | (iota16 == 15)
            old = plsc.load_gather(den_b, [sk])
            plsc.store_scatter(den_b, [sk], old + sc, mask=bd)

        def _scale(i, _):
            sv = plsc.load_gather(s_b, [jnp.full((16,), i, jnp.int32)])
            for j in range(H // 16):
                sl = pl.ds(16 * j, 16)
                xl_b[i, sl] = xl_b[i, sl] * sv
            return 0
        lax.fori_loop(0, C, _scale, 0, unroll=2)

        pltpu.sync_copy(xl_b, acc_out.at[dst_b], add=True)
        return 0
    lax.fori_loop(0, CHUNKS, _chunk, 0)

    plsc.subcore_barrier()
    pltpu.sync_copy(acc_out.at[pl.ds(row0, rows_per_tile)],
                    out_hbm.at[cid, pl.ds(row0, rows_per_tile)])
    pltpu.sync_copy(den_b, den_hbm.at[cid * NS + sid])


_sc_edge = pl.kernel(
    _sc_edge_body,
    out_type=(
        jax.ShapeDtypeStruct((NC, N_PAD, H), _f32),
        jax.ShapeDtypeStruct((NW, N_PAD), _f32),
    ),
    mesh=plsc.VectorSubcoreMesh(core_axis_name="c", subcore_axis_name="s"),
    compiler_params=pltpu.CompilerParams(needs_layout_passes=False),
    scratch_types=[
        pltpu.VMEM((C,), jnp.int32),
        pltpu.VMEM((C,), jnp.int32),
        pltpu.VMEM((C, H), _f32),
        pltpu.VMEM((C, H), _f32),
        pltpu.VMEM((C, H), _f32),
        pltpu.VMEM((C,), _f32),
        pltpu.VMEM((N_PAD,), _f32),
        pltpu.VMEM((H,), _f32),
        pltpu.VMEM_SHARED((N_PAD, H), _f32),
        pltpu.SemaphoreType.DMA,
        pltpu.SemaphoreType.DMA,
        pltpu.SemaphoreType.DMA,
    ],
)


def _easum_body(ea_ref, out_ref):
    i = pl.program_id(0)

    @pl.when(i == 0)
    def _():
        out_ref[...] = jnp.zeros_like(out_ref)
    out_ref[0, :] += jnp.sum(ea_ref[...], axis=0)


def _easum(ea):
    blk = 4000
    return pl.pallas_call(
        _easum_body,
        grid=(E // blk,),
        in_specs=[pl.BlockSpec((blk, ED), lambda i: (i, 0))],
        out_specs=pl.BlockSpec((1, ED), lambda i: (0, 0)),
        out_shape=jax.ShapeDtypeStruct((1, ED), _f32),
    )(ea)


def _edges_body(ea_ref, we1_ref, we2_ref, e1_ref, e2_ref):
    blk = ea_ref[...]
    e1_ref[...] = jnp.dot(blk, we1_ref[...], preferred_element_type=_f32)
    e2_ref[...] = jnp.dot(blk, we2_ref[...], preferred_element_type=_f32)


def _edges(ea_full, we1, we2):
    blk = 2048
    return pl.pallas_call(
        _edges_body,
        grid=(E_PAD // blk,),
        in_specs=[
            pl.BlockSpec((blk, ED), lambda i: (i, 0)),
            pl.BlockSpec((ED, H), lambda i: (0, 0)),
            pl.BlockSpec((ED, H), lambda i: (0, 0)),
        ],
        out_specs=[
            pl.BlockSpec((blk, H), lambda i: (i, 0)),
            pl.BlockSpec((blk, H), lambda i: (i, 0)),
        ],
        out_shape=[
            jax.ShapeDtypeStruct((E_PAD, H), _f32),
            jax.ShapeDtypeStruct((E_PAD, H), _f32),
        ],
    )(ea_full, we1, we2)


def _nodes_body(x_ref, wl_ref, wr_ref, bl_ref, br_ref, xl_ref, xr_ref):
    xv = x_ref[...]
    xl_ref[...] = jnp.dot(xv, wl_ref[...], preferred_element_type=_f32) \
        + bl_ref[...]
    xr_ref[...] = jnp.dot(xv, wr_ref[...], preferred_element_type=_f32) \
        + br_ref[...]


def _nodes(x, wl, wr, bl, br):
    return pl.pallas_call(
        _nodes_body,
        out_shape=[
            jax.ShapeDtypeStruct((N, H), _f32),
            jax.ShapeDtypeStruct((N, H), _f32),
        ],
    )(x, wl, wr, bl.reshape(1, H), br.reshape(1, H))


def _combine(op_ref, dp_ref, bias_ref):
    o = op_ref[0] + op_ref[1]
    d = jnp.sum(dp_ref[...], axis=0)
    return o[:N] / (d[:N, None] + 1e-16) + bias_ref[...]


def _mid_body(op_ref, dp_ref, bias_ref, g_ref, b_ref, wl_ref, wr_ref,
              bl_ref, br_ref, xl_ref, xr_ref):
    h = _combine(op_ref, dp_ref, bias_ref)
    mu = jnp.mean(h, axis=0, keepdims=True)
    var = jnp.mean((h - mu) ** 2, axis=0, keepdims=True)
    h = (h - mu) / jnp.sqrt(var + 1e-5) * g_ref[...] + b_ref[...]
    h = jnp.maximum(h, 0.0)
    xl_ref[...] = jnp.dot(h, wl_ref[...], preferred_element_type=_f32) \
        + bl_ref[...]
    xr_ref[...] = jnp.dot(h, wr_ref[...], preferred_element_type=_f32) \
        + br_ref[...]


def _mid(op, dp, bias, g, b, wl, wr, bl, br):
    return pl.pallas_call(
        _mid_body,
        out_shape=[
            jax.ShapeDtypeStruct((N, H), _f32),
            jax.ShapeDtypeStruct((N, H), _f32),
        ],
    )(op, dp, bias.reshape(1, H), g.reshape(1, H), b.reshape(1, H),
      wl, wr, bl.reshape(1, H), br.reshape(1, H))


def _post_body(op_ref, dp_ref, bias_ref, batch_ref, fcw_ref, fcb_ref, y_ref):
    h = _combine(op_ref, dp_ref, bias_ref)
    bvec = batch_ref[0, :]
    gi = lax.broadcasted_iota(jnp.int32, (G, N), 0)
    oneh = (gi == bvec[None, :]).astype(_f32)
    ps = jnp.dot(oneh, h, preferred_element_type=_f32,
                 precision=lax.Precision.HIGHEST)
    cnt = jnp.sum(oneh, axis=1)
    pooled = ps / jnp.maximum(cnt, 1.0)[:, None]
    y_ref[...] = jnp.dot(pooled, fcw_ref[...], preferred_element_type=_f32) \
        + fcb_ref[...]


def _post(op, dp, bias, batch2d, fcw, fcb):
    return pl.pallas_call(
        _post_body,
        out_shape=jax.ShapeDtypeStruct((G, 1), _f32),
    )(op, dp, bias.reshape(1, H), batch2d, fcw, fcb.reshape(1, 1))


def kernel(x, edge_attr, params, edge_index, batch):
    p1 = params['conv0']
    p2 = params['conv1']
    pad = E_PAD - E_ALL
    loop = jnp.arange(N, dtype=jnp.int32)
    zpad = jnp.zeros((pad,), jnp.int32)
    src = jnp.concatenate([edge_index[0].astype(jnp.int32), loop, zpad])
    dst = jnp.concatenate([edge_index[1].astype(jnp.int32), loop, zpad])

    ea_sum = _easum(edge_attr)
    ea_mean = ea_sum[0] / E
    ea_full = jnp.concatenate(
        [edge_attr, jnp.broadcast_to(ea_mean, (N, ED)),
         jnp.zeros((pad, ED), _f32)], axis=0)
    e1, e2 = _edges(ea_full, p1['We'], p2['We'])

    xl1, xr1 = _nodes(x, p1['Wl'], p1['Wr'], p1['bl'], p1['br'])
    op1, dp1 = _sc_edge(xl1, xr1, e1, src, dst, p1['att'])
    xl2, xr2 = _mid(op1, dp1, p1['bias'], params['bn0_g'], params['bn0_b'],
                    p2['Wl'], p2['Wr'], p2['bl'], p2['br'])
    op2, dp2 = _sc_edge(xl2, xr2, e2, src, dst, p2['att'])
    batch2d = batch.astype(jnp.int32).reshape(1, N)
    return _post(op2, dp2, p2['bias'], batch2d, params['fc_w'], params['fc_b'])
